# baseline (speedup 1.0000x reference)
import jax
import jax.numpy as jnp
from jax import lax
from jax.experimental import pallas as pl
from jax.experimental.pallas import tpu as pltpu
from jax.experimental.pallas import tpu_sc as plsc

BATCH = 16384
HIST = 50
EMBED = 64

NC = 2
NS = 16
NW = NC * NS
LANES = 16

B = BATCH * HIST
OUTW = 2 * EMBED
BPW = B // NW
BBLK = BATCH // NW
GROUP = 512
NGROUP = BPW // GROUP
NBUF = 2

assert (NGROUP - NBUF) % NBUF == 0


def _body(table_hbm, idxT_hbm, out_hbm, idx_hs, idx_bs, *rest):
    bufs = rest[:NBUF]
    isem = rest[NBUF]
    gsems = rest[NBUF + 1:2 * NBUF + 1]
    osems = rest[2 * NBUF + 1:3 * NBUF + 1]

    wid = lax.axis_index("s") * NC + lax.axis_index("c")
    base = wid * BPW
    b0 = wid * BBLK

    pltpu.async_copy(idxT_hbm.at[:, pl.ds(b0, BBLK)], idx_hs, isem).wait()

    @pl.loop(0, BPW // LANES)
    def _(j):
        k = j * LANES + lax.broadcasted_iota(jnp.int32, (LANES,), 0)
        h = lax.rem(k, HIST)
        b = lax.div(k, HIST)
        idx_bs[pl.ds(j * LANES, LANES)] = plsc.load_gather(idx_hs, [h, b])

    def start_gather(g, i):
        pltpu.async_copy(table_hbm.at[idx_bs.at[pl.ds(g * GROUP, GROUP)]],
                         bufs[i], gsems[i])

    def wait_gather(i):
        pltpu.make_async_copy(table_hbm.at[idx_bs.at[pl.ds(0, GROUP)]],
                              bufs[i], gsems[i]).wait()

    def start_out(g, i):
        pltpu.async_copy(bufs[i],
                         out_hbm.at[pl.ds(base + g * GROUP, GROUP),
                                    pl.ds(0, EMBED)], osems[i])

    def wait_out(i):
        pltpu.make_async_copy(bufs[i],
                              out_hbm.at[pl.ds(base, GROUP), pl.ds(0, EMBED)],
                              osems[i]).wait()

    for i in range(NBUF):
        start_gather(i, i)

    @pl.loop(0, NGROUP - NBUF, step=NBUF)
    def _(g):
        for i in range(NBUF):
            wait_gather(i)
            start_out(g + i, i)
        for i in range(NBUF):
            wait_out(i)
            start_gather(g + NBUF + i, i)

    for i in range(NBUF):
        wait_gather(i)
        start_out(NGROUP - NBUF + i, i)
    for i in range(NBUF):
        wait_out(i)


@jax.jit
def _lookup(table, idxT):
    mesh = plsc.VectorSubcoreMesh(core_axis_name="c", subcore_axis_name="s")
    f = pl.kernel(
        _body,
        out_type=jax.ShapeDtypeStruct((B, OUTW), jnp.float32),
        mesh=mesh,
        compiler_params=pltpu.CompilerParams(use_tc_tiling_on_sc=False,
                                             needs_layout_passes=False),
        scratch_types=(
            [pltpu.VMEM((HIST, BBLK), jnp.int32),
             pltpu.VMEM((BPW,), jnp.int32)]
            + [pltpu.VMEM((GROUP, EMBED), jnp.float32)] * NBUF
            + [pltpu.SemaphoreType.DMA] * (2 * NBUF + 1)
        ),
    )
    return f(table, idxT)


def kernel(input_x, table):
    idxT = input_x.T.astype(jnp.int32)
    out_p = _lookup(table, idxT)
    return out_p[:, :EMBED].reshape(BATCH, HIST, EMBED)


# speedup vs baseline: 1.0755x; 1.0755x over previous
import jax
import jax.numpy as jnp
from jax import lax
from jax.experimental import pallas as pl
from jax.experimental.pallas import tpu as pltpu
from jax.experimental.pallas import tpu_sc as plsc

BATCH = 16384
HIST = 50
EMBED = 64

NC = 2
NS = 16
NW = NC * NS

B = BATCH * HIST
BPW = B // NW
CHUNK = 512
GROUP = 512
CPG = GROUP // CHUNK
NGROUP = BPW // GROUP


def _body(table_hbm, idx_hbm, out_hbm, idx_v, buf0, buf1,
          gsem0, gsem1, osem0, osem1):
    wid = lax.axis_index("s") * NC + lax.axis_index("c")
    base = wid * BPW

    pltpu.sync_copy(idx_hbm.at[wid], idx_v)

    def start_gathers(g, buf, sem):
        for c in range(CPG):
            pltpu.async_copy(
                table_hbm.at[idx_v.at[g * CPG + c]],
                buf.at[pl.ds(c * CHUNK, CHUNK)],
                sem,
            )

    def wait_gathers(buf, sem):
        for c in range(CPG):
            pltpu.make_async_copy(
                table_hbm.at[idx_v.at[c]],
                buf.at[pl.ds(c * CHUNK, CHUNK)],
                sem,
            ).wait()

    def start_out(g, buf, sem):
        pltpu.async_copy(buf, out_hbm.at[pl.ds(base + g * GROUP, GROUP)], sem)

    def wait_out(buf, sem):
        pltpu.make_async_copy(
            buf, out_hbm.at[pl.ds(base, GROUP)], sem
        ).wait()

    start_gathers(0, buf0, gsem0)
    start_gathers(1, buf1, gsem1)

    @pl.loop(0, NGROUP - 2, step=2)
    def _(g):
        wait_gathers(buf0, gsem0)
        start_out(g, buf0, osem0)
        wait_gathers(buf1, gsem1)
        start_out(g + 1, buf1, osem1)
        wait_out(buf0, osem0)
        start_gathers(g + 2, buf0, gsem0)
        wait_out(buf1, osem1)
        start_gathers(g + 3, buf1, gsem1)

    g_last = NGROUP - 2
    wait_gathers(buf0, gsem0)
    start_out(g_last, buf0, osem0)
    wait_gathers(buf1, gsem1)
    start_out(g_last + 1, buf1, osem1)
    wait_out(buf0, osem0)
    wait_out(buf1, osem1)


@jax.jit
def _lookup(table, idx):
    mesh = plsc.VectorSubcoreMesh(core_axis_name="c", subcore_axis_name="s")
    f = pl.kernel(
        _body,
        out_type=jax.ShapeDtypeStruct((B, EMBED), jnp.float32),
        mesh=mesh,
        compiler_params=pltpu.CompilerParams(use_tc_tiling_on_sc=False),
        scratch_types=[
            pltpu.VMEM((NGROUP * CPG, CHUNK), jnp.int32),
            pltpu.VMEM((GROUP, EMBED), jnp.float32),
            pltpu.VMEM((GROUP, EMBED), jnp.float32),
            pltpu.SemaphoreType.DMA,
            pltpu.SemaphoreType.DMA,
            pltpu.SemaphoreType.DMA,
            pltpu.SemaphoreType.DMA,
        ],
    )
    return f(table, idx)


def kernel(input_x, table):
    idx = input_x.reshape(NW, NGROUP * CPG, CHUNK).astype(jnp.int32)
    out = _lookup(table, idx)
    return out.reshape(BATCH, HIST, EMBED)
